# baseline (speedup 1.0000x reference)
import functools

import jax
import jax.numpy as jnp
from jax import lax
from jax.experimental import pallas as pl
from jax.experimental.pallas import tpu as pltpu
from jax.experimental.pallas import tpu_sc as plsc

_LANES = 16
_FPW = 2


def _body(nc, table_hbm, words_hbm, out_hbm, trow, ibuf, obuf, isem, osem):
    hist, batch = words_hbm.shape
    wid = lax.axis_index("s") * nc + lax.axis_index("c")
    nvec = batch // _LANES

    def iload(h, slot):
        return pltpu.make_async_copy(words_hbm.at[h], ibuf.at[slot], isem.at[slot])

    for f in range(_FPW):
        d = wid * _FPW + f
        pltpu.sync_copy(table_hbm.at[d], trow)
        head = trow[pl.ds(0, _LANES)]
        trow[pl.ds(0, _LANES)] = jnp.where(
            lax.iota(jnp.int32, _LANES) == 0, jnp.float32(0.0), head
        )

        def ostore(h, slot, d=d):
            return pltpu.make_async_copy(
                obuf.at[slot], out_hbm.at[h, d], osem.at[slot]
            )

        iload(0, 0).start()

        def hstep(h, carry, d=d, ostore=ostore):
            slot = lax.rem(h, 2)

            @pl.when(h >= 2)
            def _():
                ostore(h - 2, slot).wait()

            iload(h, slot).wait()

            @pl.when(h + 1 < hist)
            def _():
                iload(h + 1, 1 - slot).start()

            def vstep(i, c):
                iv = ibuf[slot, pl.ds(i * _LANES, _LANES)]
                obuf[slot, pl.ds(i * _LANES, _LANES)] = plsc.load_gather(
                    trow, [iv]
                )
                return c

            lax.fori_loop(0, nvec, vstep, 0)
            ostore(h, slot).start()
            return carry

        lax.fori_loop(0, hist, hstep, 0)
        ostore(hist - 2, lax.rem(hist - 2, 2)).wait()
        ostore(hist - 1, lax.rem(hist - 1, 2)).wait()


def kernel(words, table):
    B, H = words.shape
    V, D = table.shape
    info = plsc.get_sparse_core_info()
    nc, ns = info.num_cores, info.num_subcores

    mesh = plsc.VectorSubcoreMesh(core_axis_name="c", subcore_axis_name="s")
    run = pl.kernel(
        functools.partial(_body, nc),
        out_type=jax.ShapeDtypeStruct((H, D, B), jnp.float32),
        mesh=mesh,
        compiler_params=pltpu.CompilerParams(
            use_tc_tiling_on_sc=False, needs_layout_passes=False
        ),
        scratch_types=[
            pltpu.VMEM((V,), jnp.float32),
            pltpu.VMEM((2, B), jnp.int32),
            pltpu.VMEM((2, B), jnp.float32),
            pltpu.SemaphoreType.DMA((2,)),
            pltpu.SemaphoreType.DMA((2,)),
        ],
    )
    out3 = run(table.T, words.T.astype(jnp.int32))
    return out3.transpose(2, 0, 1)


# speedup vs baseline: 1.8011x; 1.8011x over previous
import functools

import jax
import jax.numpy as jnp
from jax import lax
from jax.experimental import pallas as pl
from jax.experimental.pallas import tpu as pltpu
from jax.experimental.pallas import tpu_sc as plsc

_LANES = 16
_FPW = 2
_UNROLL = 8


def _body(nc, table_hbm, words_hbm, out_hbm, trow, ibuf, obuf, isem, osem):
    hist, batch = words_hbm.shape
    wid = lax.axis_index("s") * nc + lax.axis_index("c")
    nvec = batch // _LANES

    def iload(h, slot):
        return pltpu.make_async_copy(words_hbm.at[h], ibuf.at[slot], isem.at[slot])

    for f in range(_FPW):
        d = wid * _FPW + f
        pltpu.sync_copy(table_hbm.at[d], trow)
        head = trow[pl.ds(0, _LANES)]
        trow[pl.ds(0, _LANES)] = jnp.where(
            lax.iota(jnp.int32, _LANES) == 0, jnp.float32(0.0), head
        )

        def ostore(h, slot, d=d):
            return pltpu.make_async_copy(
                obuf.at[slot], out_hbm.at[h, d], osem.at[slot]
            )

        iload(0, 0).start()

        def hstep(h, carry, d=d, ostore=ostore):
            slot = lax.rem(h, 2)

            @pl.when(h >= 2)
            def _():
                ostore(h - 2, slot).wait()

            iload(h, slot).wait()

            @pl.when(h + 1 < hist)
            def _():
                iload(h + 1, 1 - slot).start()

            def vstep(g, c):
                base = g * (_UNROLL * _LANES)
                for u in range(_UNROLL):
                    off = base + u * _LANES
                    iv = ibuf[slot, pl.ds(off, _LANES)]
                    obuf[slot, pl.ds(off, _LANES)] = plsc.load_gather(
                        trow, [iv]
                    )
                return c

            lax.fori_loop(0, nvec // _UNROLL, vstep, 0)
            ostore(h, slot).start()
            return carry

        lax.fori_loop(0, hist, hstep, 0)
        ostore(hist - 2, lax.rem(hist - 2, 2)).wait()
        ostore(hist - 1, lax.rem(hist - 1, 2)).wait()


def kernel(words, table):
    B, H = words.shape
    V, D = table.shape
    info = plsc.get_sparse_core_info()
    nc, ns = info.num_cores, info.num_subcores

    mesh = plsc.VectorSubcoreMesh(core_axis_name="c", subcore_axis_name="s")
    run = pl.kernel(
        functools.partial(_body, nc),
        out_type=jax.ShapeDtypeStruct((H, D, B), jnp.float32),
        mesh=mesh,
        compiler_params=pltpu.CompilerParams(
            use_tc_tiling_on_sc=False, needs_layout_passes=False
        ),
        scratch_types=[
            pltpu.VMEM((V + (-V) % 128,), jnp.float32),
            pltpu.VMEM((2, B), jnp.int32),
            pltpu.VMEM((2, B), jnp.float32),
            pltpu.SemaphoreType.DMA((2,)),
            pltpu.SemaphoreType.DMA((2,)),
        ],
    )
    vpad = (-V) % 128
    tpad = jnp.concatenate(
        [table.T, jnp.zeros((D, vpad), table.dtype)], axis=1
    )
    out3 = run(tpad, words.T.astype(jnp.int32))
    return out3.transpose(2, 0, 1)
